# Initial kernel scaffold; baseline (speedup 1.0000x reference)
import jax
import jax.numpy as jnp
from jax import lax
from jax.experimental import pallas as pl
from jax.experimental.pallas import tpu as pltpu
from jax.experimental.pallas import tpu_sc as plsc

NC = 2
NS = 16
L = 16


def _proj_call(s, v, Wq, bq2, Wkv, bkv2):
    N, DIM = s.shape

    def body(s_ref, v_ref, wq_ref, bq_ref, wkv_ref, bkv_ref,
             q_ref, k_ref, x0_ref, x1_ref, x2_ref, x3_ref):
        sblk = s_ref[...]
        q_ref[...] = (
            jnp.dot(sblk, wq_ref[...], preferred_element_type=jnp.float32)
            + bq_ref[...])
        kv = (jnp.dot(sblk, wkv_ref[...], preferred_element_type=jnp.float32)
              + bkv_ref[...])
        k_ref[...] = kv[:, :DIM]
        x0_ref[...] = kv[:, DIM:2 * DIM]
        vv = kv[:, 2 * DIM:]
        x1_ref[...] = vv * v_ref[:, 0, :]
        x2_ref[...] = vv * v_ref[:, 1, :]
        x3_ref[...] = vv * v_ref[:, 2, :]

    BLK = 1000
    grid = (N // BLK,)
    out2d = pl.BlockSpec((BLK, DIM), lambda i: (i, 0))
    return pl.pallas_call(
        body,
        grid=grid,
        in_specs=[
            pl.BlockSpec((BLK, DIM), lambda i: (i, 0)),
            pl.BlockSpec((BLK, 3, DIM), lambda i: (i, 0, 0)),
            pl.BlockSpec((DIM, DIM), lambda i: (0, 0)),
            pl.BlockSpec((1, DIM), lambda i: (0, 0)),
            pl.BlockSpec((DIM, 3 * DIM), lambda i: (0, 0)),
            pl.BlockSpec((1, 3 * DIM), lambda i: (0, 0)),
        ],
        out_specs=[out2d] * 6,
        out_shape=[jax.ShapeDtypeStruct((N, DIM), jnp.float32)] * 6,
    )(s, v, Wq, bq2, Wkv, bkv2)


def _edge_w_call(row, col, q, k):
    (E,) = row.shape
    N, DIM = q.shape
    NW = NC * NS
    EPW = E // NW
    C = 80
    NCHUNK = EPW // C
    NSLC = DIM // L

    mesh = plsc.VectorSubcoreMesh(core_axis_name="c", subcore_axis_name="s",
                                  num_cores=NC, num_subcores=NS)

    def body(row_hbm, col_hbm, q_hbm, k_hbm, w_hbm,
             idxr, idxc, qrows, krows, wbuf, sem):
        cid = lax.axis_index("c")
        sid = lax.axis_index("s")
        wid = sid * NC + cid
        base = wid * EPW
        lane = lax.iota(jnp.int32, L)

        def chunk(i, carry):
            off = base + i * C
            pltpu.sync_copy(row_hbm.at[pl.ds(off, C)], idxr)
            pltpu.sync_copy(col_hbm.at[pl.ds(off, C)], idxc)
            cp1 = pltpu.async_copy(q_hbm.at[idxr], qrows, sem)
            cp2 = pltpu.async_copy(k_hbm.at[idxc], krows, sem)
            cp1.wait()
            cp2.wait()
            for g in range(C // L):
                wvec = jnp.zeros((L,), jnp.float32)
                for e16 in range(L):
                    e = g * L + e16
                    acc = qrows[e, pl.ds(0, L)] * krows[e, pl.ds(0, L)]
                    for jj in range(1, NSLC):
                        acc = acc + (qrows[e, pl.ds(jj * L, L)]
                                     * krows[e, pl.ds(jj * L, L)])
                    we = jnp.sum(acc, axis=0)
                    wvec = jnp.where(lane == e16, we, wvec)
                wbuf[pl.ds(g * L, L)] = wvec
            pltpu.sync_copy(wbuf, w_hbm.at[pl.ds(off, C)])
            return carry

        lax.fori_loop(0, NCHUNK, chunk, 0)

    return pl.kernel(
        body,
        out_type=jax.ShapeDtypeStruct((E,), jnp.float32),
        mesh=mesh,
        scratch_types=[
            pltpu.VMEM((C,), jnp.int32),
            pltpu.VMEM((C,), jnp.int32),
            pltpu.VMEM((C, DIM), jnp.float32),
            pltpu.VMEM((C, DIM), jnp.float32),
            pltpu.VMEM((C,), jnp.float32),
            pltpu.SemaphoreType.DMA,
        ],
    )(row, col, q, k)


def _scatter_call(row, col, w, x0, x1, x2, x3, zeros):
    (E,) = row.shape
    N, DIM = x0.shape
    EPT = E // NS
    RPT = N // NS
    C = 80
    NCHUNK = EPT // C
    NSLC = DIM // L

    mesh = plsc.VectorSubcoreMesh(core_axis_name="c", subcore_axis_name="s",
                                  num_cores=NC, num_subcores=NS)

    def body(row_hbm, col_hbm, w_hbm, x0_hbm, x1_hbm, x2_hbm, x3_hbm, z_hbm,
             o0, o1, o2, o3, idxr, idxc, wv, xrows, msg, accum, sem):
        cid = lax.axis_index("c")
        sid = lax.axis_index("s")

        def one_quarter(x_hbm, out_hbm):
            pltpu.sync_copy(z_hbm.at[pl.ds(sid * RPT, RPT)],
                            accum.at[pl.ds(sid * RPT, RPT)])
            plsc.subcore_barrier()

            def chunk(i, carry):
                off = sid * EPT + i * C
                pltpu.sync_copy(row_hbm.at[pl.ds(off, C)], idxr)
                pltpu.sync_copy(col_hbm.at[pl.ds(off, C)], idxc)
                pltpu.sync_copy(w_hbm.at[pl.ds(off, C)], wv)
                pltpu.async_copy(x_hbm.at[idxc], xrows, sem).wait()
                for e in range(C):
                    we = wv[e]
                    for jj in range(NSLC):
                        msg[e, pl.ds(jj * L, L)] = (
                            we * xrows[e, pl.ds(jj * L, L)])
                pltpu.sync_copy(msg, accum.at[idxr], add=True)
                return carry

            lax.fori_loop(0, NCHUNK, chunk, 0)
            plsc.subcore_barrier()
            pltpu.sync_copy(accum.at[pl.ds(sid * RPT, RPT)],
                            out_hbm.at[pl.ds(sid * RPT, RPT)])

        @pl.when(cid == 0)
        def _():
            one_quarter(x0_hbm, o0)
            one_quarter(x1_hbm, o1)

        @pl.when(cid == 1)
        def _():
            one_quarter(x2_hbm, o2)
            one_quarter(x3_hbm, o3)

    return pl.kernel(
        body,
        out_type=[jax.ShapeDtypeStruct((N, DIM), jnp.float32)] * 4,
        mesh=mesh,
        scratch_types=[
            pltpu.VMEM((C,), jnp.int32),
            pltpu.VMEM((C,), jnp.int32),
            pltpu.VMEM((C,), jnp.float32),
            pltpu.VMEM((C, DIM), jnp.float32),
            pltpu.VMEM((C, DIM), jnp.float32),
            pltpu.VMEM_SHARED((N, DIM), jnp.float32),
            pltpu.SemaphoreType.DMA,
        ],
    )(row, col, w, x0, x1, x2, x3, zeros)


def _finish_call(s, v, a0, a1, a2, a3, Wm, bm2):
    N, DIM = s.shape

    def body(s_ref, v_ref, a0_ref, a1_ref, a2_ref, a3_ref, wm_ref, bm_ref,
             so_ref, vo_ref):
        so_ref[...] = s_ref[...] + a0_ref[...]
        wm = wm_ref[...]
        for j, aj in enumerate((a1_ref, a2_ref, a3_ref)):
            vo_ref[:, j, :] = (
                v_ref[:, j, :]
                + jnp.dot(aj[...], wm, preferred_element_type=jnp.float32)
                + bm_ref[...])

    BLK = 1000
    grid = (N // BLK,)
    blk2d = pl.BlockSpec((BLK, DIM), lambda i: (i, 0))
    return pl.pallas_call(
        body,
        grid=grid,
        in_specs=[
            blk2d,
            pl.BlockSpec((BLK, 3, DIM), lambda i: (i, 0, 0)),
            blk2d, blk2d, blk2d, blk2d,
            pl.BlockSpec((DIM, DIM), lambda i: (0, 0)),
            pl.BlockSpec((1, DIM), lambda i: (0, 0)),
        ],
        out_specs=[
            blk2d,
            pl.BlockSpec((BLK, 3, DIM), lambda i: (i, 0, 0)),
        ],
        out_shape=[
            jax.ShapeDtypeStruct((N, DIM), jnp.float32),
            jax.ShapeDtypeStruct((N, 3, DIM), jnp.float32),
        ],
    )(s, v, a0, a1, a2, a3, Wm, bm2)


def kernel(s, v, edge_index, Wq, bq, Wkv, bkv, Wm, bm):
    N, DIM = s.shape
    row = edge_index[0]
    col = edge_index[1]
    q, k, x0, x1, x2, x3 = _proj_call(
        s, v, Wq, bq.reshape(1, -1), Wkv, bkv.reshape(1, -1))
    w = _edge_w_call(row, col, q, k)
    zeros = jnp.zeros((N, DIM), jnp.float32)
    a0, a1, a2, a3 = _scatter_call(row, col, w, x0, x1, x2, x3, zeros)
    return _finish_call(s, v, a0, a1, a2, a3, Wm, bm.reshape(1, -1))


# speedup vs baseline: 15.3063x; 15.3063x over previous
import jax
import jax.numpy as jnp
from jax import lax
from jax.experimental import pallas as pl
from jax.experimental.pallas import tpu as pltpu
from jax.experimental.pallas import tpu_sc as plsc

NC = 2
NS = 16
L = 16


def _proj_call(s, v, Wq, bq2, Wkv, bkv2):
    N, DIM = s.shape

    def body(s_ref, v_ref, wq_ref, bq_ref, wkv_ref, bkv_ref,
             q_ref, k_ref, x0_ref, x1_ref, x2_ref, x3_ref):
        sblk = s_ref[...]
        q_ref[...] = (
            jnp.dot(sblk, wq_ref[...], preferred_element_type=jnp.float32)
            + bq_ref[...])
        kv = (jnp.dot(sblk, wkv_ref[...], preferred_element_type=jnp.float32)
              + bkv_ref[...])
        k_ref[...] = kv[:, :DIM]
        x0_ref[...] = kv[:, DIM:2 * DIM]
        vv = kv[:, 2 * DIM:]
        x1_ref[...] = vv * v_ref[:, 0, :]
        x2_ref[...] = vv * v_ref[:, 1, :]
        x3_ref[...] = vv * v_ref[:, 2, :]

    BLK = 1000
    grid = (N // BLK,)
    out2d = pl.BlockSpec((BLK, DIM), lambda i: (i, 0))
    return pl.pallas_call(
        body,
        grid=grid,
        in_specs=[
            pl.BlockSpec((BLK, DIM), lambda i: (i, 0)),
            pl.BlockSpec((BLK, 3, DIM), lambda i: (i, 0, 0)),
            pl.BlockSpec((DIM, DIM), lambda i: (0, 0)),
            pl.BlockSpec((1, DIM), lambda i: (0, 0)),
            pl.BlockSpec((DIM, 3 * DIM), lambda i: (0, 0)),
            pl.BlockSpec((1, 3 * DIM), lambda i: (0, 0)),
        ],
        out_specs=[out2d] * 6,
        out_shape=[jax.ShapeDtypeStruct((N, DIM), jnp.float32)] * 6,
    )(s, v, Wq, bq2, Wkv, bkv2)


def _edge_w_call(row, col, q, k):
    (E,) = row.shape
    N, DIM = q.shape
    NW = NC * NS
    EPW = E // NW
    C = 80
    NCHUNK = EPW // C
    NSLC = DIM // L

    mesh = plsc.VectorSubcoreMesh(core_axis_name="c", subcore_axis_name="s",
                                  num_cores=NC, num_subcores=NS)

    def body(row_hbm, col_hbm, q_hbm, k_hbm, w_hbm,
             idxr, idxc, qrows, krows, wbuf, sem):
        cid = lax.axis_index("c")
        sid = lax.axis_index("s")
        wid = sid * NC + cid
        base = wid * EPW
        lane = lax.iota(jnp.int32, L)

        def chunk(i, carry):
            off = base + i * C
            pltpu.sync_copy(row_hbm.at[pl.ds(off, C)], idxr)
            pltpu.sync_copy(col_hbm.at[pl.ds(off, C)], idxc)
            cp1 = pltpu.async_copy(q_hbm.at[idxr], qrows, sem)
            cp2 = pltpu.async_copy(k_hbm.at[idxc], krows, sem)
            cp1.wait()
            cp2.wait()
            for g in range(C // L):
                wvec = jnp.zeros((L,), jnp.float32)
                for e16 in range(L):
                    e = g * L + e16
                    acc = qrows[e, pl.ds(0, L)] * krows[e, pl.ds(0, L)]
                    for jj in range(1, NSLC):
                        acc = acc + (qrows[e, pl.ds(jj * L, L)]
                                     * krows[e, pl.ds(jj * L, L)])
                    we = jnp.sum(acc, axis=0)
                    wvec = jnp.where(lane == e16, we, wvec)
                wbuf[pl.ds(g * L, L)] = wvec
            pltpu.sync_copy(wbuf, w_hbm.at[pl.ds(off, C)])
            return carry

        lax.fori_loop(0, NCHUNK, chunk, 0)

    return pl.kernel(
        body,
        out_type=jax.ShapeDtypeStruct((E,), jnp.float32),
        mesh=mesh,
        compiler_params=pltpu.CompilerParams(needs_layout_passes=False),
        scratch_types=[
            pltpu.VMEM((C,), jnp.int32),
            pltpu.VMEM((C,), jnp.int32),
            pltpu.VMEM((C, DIM), jnp.float32),
            pltpu.VMEM((C, DIM), jnp.float32),
            pltpu.VMEM((C,), jnp.float32),
            pltpu.SemaphoreType.DMA,
        ],
    )(row, col, q, k)


def _scatter_call(row, col, w, x0, x1, x2, x3, zeros):
    (E,) = row.shape
    N, DIM = x0.shape
    NP = zeros.shape[0]
    EPT = E // NS
    RPT = NP // NS
    C = 80
    NCHUNK = EPT // C
    NSLC = DIM // L

    mesh = plsc.VectorSubcoreMesh(core_axis_name="c", subcore_axis_name="s",
                                  num_cores=NC, num_subcores=NS)

    def body(row_hbm, col_hbm, w_hbm, x0_hbm, x1_hbm, x2_hbm, x3_hbm, z_hbm,
             o0, o1, o2, o3, idxr, idxc, wv, xrows, msg, accum, sem):
        cid = lax.axis_index("c")
        sid = lax.axis_index("s")

        def one_quarter(x_hbm, out_hbm):
            pltpu.sync_copy(z_hbm.at[pl.ds(sid * RPT, RPT)],
                            accum.at[pl.ds(sid * RPT, RPT)])
            plsc.subcore_barrier()

            def chunk(i, carry):
                off = sid * EPT + i * C
                pltpu.sync_copy(row_hbm.at[pl.ds(off, C)], idxr)
                pltpu.sync_copy(col_hbm.at[pl.ds(off, C)], idxc)
                pltpu.sync_copy(w_hbm.at[pl.ds(off, C)], wv)
                pltpu.async_copy(x_hbm.at[idxc], xrows, sem).wait()
                for g in range(C // L):
                    wg = wv[pl.ds(g * L, L)]
                    for e16 in range(L):
                        e = g * L + e16
                        we = wg[e16]
                        for jj in range(NSLC):
                            msg[e, pl.ds(jj * L, L)] = (
                                we * xrows[e, pl.ds(jj * L, L)])
                pltpu.sync_copy(msg, accum.at[idxr], add=True)
                return carry

            lax.fori_loop(0, NCHUNK, chunk, 0)
            plsc.subcore_barrier()
            pltpu.sync_copy(accum.at[pl.ds(sid * RPT, RPT)],
                            out_hbm.at[pl.ds(sid * RPT, RPT)])

        @pl.when(cid == 0)
        def _():
            one_quarter(x0_hbm, o0)
            one_quarter(x1_hbm, o1)

        @pl.when(cid == 1)
        def _():
            one_quarter(x2_hbm, o2)
            one_quarter(x3_hbm, o3)

    return pl.kernel(
        body,
        out_type=[jax.ShapeDtypeStruct((NP, DIM), jnp.float32)] * 4,
        mesh=mesh,
        compiler_params=pltpu.CompilerParams(needs_layout_passes=False),
        scratch_types=[
            pltpu.VMEM((C,), jnp.int32),
            pltpu.VMEM((C,), jnp.int32),
            pltpu.VMEM((C,), jnp.float32),
            pltpu.VMEM((C, DIM), jnp.float32),
            pltpu.VMEM((C, DIM), jnp.float32),
            pltpu.VMEM_SHARED((NP, DIM), jnp.float32),
            pltpu.SemaphoreType.DMA,
        ],
    )(row, col, w, x0, x1, x2, x3, zeros)


def _finish_call(s, v, a0, a1, a2, a3, Wm, bm2):
    N, DIM = s.shape

    def body(s_ref, v_ref, a0_ref, a1_ref, a2_ref, a3_ref, wm_ref, bm_ref,
             so_ref, vo_ref):
        so_ref[...] = s_ref[...] + a0_ref[...]
        wm = wm_ref[...]
        for j, aj in enumerate((a1_ref, a2_ref, a3_ref)):
            vo_ref[:, j, :] = (
                v_ref[:, j, :]
                + jnp.dot(aj[...], wm, preferred_element_type=jnp.float32)
                + bm_ref[...])

    BLK = 1000
    grid = (N // BLK,)
    blk2d = pl.BlockSpec((BLK, DIM), lambda i: (i, 0))
    return pl.pallas_call(
        body,
        grid=grid,
        in_specs=[
            blk2d,
            pl.BlockSpec((BLK, 3, DIM), lambda i: (i, 0, 0)),
            blk2d, blk2d, blk2d, blk2d,
            pl.BlockSpec((DIM, DIM), lambda i: (0, 0)),
            pl.BlockSpec((1, DIM), lambda i: (0, 0)),
        ],
        out_specs=[
            blk2d,
            pl.BlockSpec((BLK, 3, DIM), lambda i: (i, 0, 0)),
        ],
        out_shape=[
            jax.ShapeDtypeStruct((N, DIM), jnp.float32),
            jax.ShapeDtypeStruct((N, 3, DIM), jnp.float32),
        ],
    )(s, v, a0, a1, a2, a3, Wm, bm2)


def kernel(s, v, edge_index, Wq, bq, Wkv, bkv, Wm, bm):
    N, DIM = s.shape
    row = edge_index[0]
    col = edge_index[1]
    q, k, x0, x1, x2, x3 = _proj_call(
        s, v, Wq, bq.reshape(1, -1), Wkv, bkv.reshape(1, -1))
    w = _edge_w_call(row, col, q, k)
    NP = ((N + 8 * NS - 1) // (8 * NS)) * (8 * NS)
    zeros = jnp.zeros((NP, DIM), jnp.float32)
    a0, a1, a2, a3 = _scatter_call(row, col, w, x0, x1, x2, x3, zeros)
    a0, a1, a2, a3 = (a[:N] for a in (a0, a1, a2, a3))
    return _finish_call(s, v, a0, a1, a2, a3, Wm, bm.reshape(1, -1))


# speedup vs baseline: 27.3249x; 1.7852x over previous
import jax
import jax.numpy as jnp
from jax import lax
from jax.experimental import pallas as pl
from jax.experimental.pallas import tpu as pltpu
from jax.experimental.pallas import tpu_sc as plsc

NC = 2
NS = 16
L = 16


def _proj_call(s, v, Wq, bq2, Wkv, bkv2):
    N, DIM = s.shape

    def body(s_ref, v_ref, wq_ref, bq_ref, wkv_ref, bkv_ref,
             q_ref, k_ref, x0_ref, x1_ref, x2_ref, x3_ref):
        sblk = s_ref[...]
        q_ref[...] = (
            jnp.dot(sblk, wq_ref[...], preferred_element_type=jnp.float32)
            + bq_ref[...])
        kv = (jnp.dot(sblk, wkv_ref[...], preferred_element_type=jnp.float32)
              + bkv_ref[...])
        k_ref[...] = kv[:, :DIM]
        x0_ref[...] = kv[:, DIM:2 * DIM]
        vv = kv[:, 2 * DIM:]
        x1_ref[...] = vv * v_ref[:, 0, :]
        x2_ref[...] = vv * v_ref[:, 1, :]
        x3_ref[...] = vv * v_ref[:, 2, :]

    BLK = 1000
    grid = (N // BLK,)
    out2d = pl.BlockSpec((BLK, DIM), lambda i: (i, 0))
    return pl.pallas_call(
        body,
        grid=grid,
        in_specs=[
            pl.BlockSpec((BLK, DIM), lambda i: (i, 0)),
            pl.BlockSpec((BLK, 3, DIM), lambda i: (i, 0, 0)),
            pl.BlockSpec((DIM, DIM), lambda i: (0, 0)),
            pl.BlockSpec((1, DIM), lambda i: (0, 0)),
            pl.BlockSpec((DIM, 3 * DIM), lambda i: (0, 0)),
            pl.BlockSpec((1, 3 * DIM), lambda i: (0, 0)),
        ],
        out_specs=[out2d] * 6,
        out_shape=[jax.ShapeDtypeStruct((N, DIM), jnp.float32)] * 6,
    )(s, v, Wq, bq2, Wkv, bkv2)


def _edge_w_call(row, col, q, k):
    (E,) = row.shape
    N, DIM = q.shape
    NW = NC * NS
    EPW = E // NW
    C = 80
    NCHUNK = EPW // C
    NSLC = DIM // L

    mesh = plsc.VectorSubcoreMesh(core_axis_name="c", subcore_axis_name="s",
                                  num_cores=NC, num_subcores=NS)

    def body(row_hbm, col_hbm, q_hbm, k_hbm, w_hbm,
             rows_v, cols_v, qrows, krows, wbuf,
             semg0, semg1, semw0, semw1):
        cid = lax.axis_index("c")
        sid = lax.axis_index("s")
        wid = sid * NC + cid
        base = wid * EPW
        lane = lax.iota(jnp.int32, L)
        semg = (semg0, semg1)
        semw = (semw0, semw1)

        pltpu.sync_copy(row_hbm.at[pl.ds(base, EPW)], rows_v)
        pltpu.sync_copy(col_hbm.at[pl.ds(base, EPW)], cols_v)

        def issue(j, b):
            idxr = rows_v.at[pl.ds(j * C, C)]
            idxc = cols_v.at[pl.ds(j * C, C)]
            pltpu.async_copy(q_hbm.at[idxr], qrows.at[b], semg[b])
            pltpu.async_copy(k_hbm.at[idxc], krows.at[b], semg[b])

        def wait_gather(b):
            dummy = rows_v.at[pl.ds(0, C)]
            pltpu.make_async_copy(q_hbm.at[dummy], qrows.at[b],
                                  semg[b]).wait()
            pltpu.make_async_copy(k_hbm.at[dummy], krows.at[b],
                                  semg[b]).wait()

        def wait_w(b):
            pltpu.make_async_copy(wbuf.at[b], w_hbm.at[pl.ds(base, C)],
                                  semw[b]).wait()

        def compute(b):
            for g in range(C // L):
                wvec = jnp.zeros((L,), jnp.float32)
                for e16 in range(L):
                    e = g * L + e16
                    acc = qrows[b, e, pl.ds(0, L)] * krows[b, e, pl.ds(0, L)]
                    for jj in range(1, NSLC):
                        acc = acc + (qrows[b, e, pl.ds(jj * L, L)]
                                     * krows[b, e, pl.ds(jj * L, L)])
                    we = jnp.sum(acc, axis=0)
                    wvec = jnp.where(lane == e16, we, wvec)
                wbuf[b, pl.ds(g * L, L)] = wvec

        issue(0, 0)

        def outer(g, carry):
            for b in range(2):
                j = 2 * g + b

                @pl.when(j < NCHUNK)
                def _():
                    @pl.when(j + 1 < NCHUNK)
                    def _():
                        issue(j + 1, 1 - b)

                    wait_gather(b)

                    @pl.when(j >= 2)
                    def _():
                        wait_w(b)

                    compute(b)
                    pltpu.async_copy(wbuf.at[b],
                                     w_hbm.at[pl.ds(base + j * C, C)],
                                     semw[b])
            return carry

        lax.fori_loop(0, (NCHUNK + 1) // 2, outer, 0)
        wait_w(0)
        wait_w(1)

    return pl.kernel(
        body,
        out_type=jax.ShapeDtypeStruct((E,), jnp.float32),
        mesh=mesh,
        compiler_params=pltpu.CompilerParams(needs_layout_passes=False),
        scratch_types=[
            pltpu.VMEM((EPW,), jnp.int32),
            pltpu.VMEM((EPW,), jnp.int32),
            pltpu.VMEM((2, C, DIM), jnp.float32),
            pltpu.VMEM((2, C, DIM), jnp.float32),
            pltpu.VMEM((2, C), jnp.float32),
            pltpu.SemaphoreType.DMA,
            pltpu.SemaphoreType.DMA,
            pltpu.SemaphoreType.DMA,
            pltpu.SemaphoreType.DMA,
        ],
    )(row, col, q, k)


def _scatter_call(row, col, w, x0, x1, x2, x3, zeros):
    (E,) = col.shape
    N, DIM = x0.shape
    NP = zeros.shape[0]
    EPT = E // NS
    RPT = NP // NS
    C = 80
    NCHQ = EPT // C
    BC = 25
    SBLK = BC * C
    NBLK = NCHQ // BC
    NSLC = DIM // L

    mesh = plsc.VectorSubcoreMesh(core_axis_name="c", subcore_axis_name="s",
                                  num_cores=NC, num_subcores=NS)

    def body(row_hbm, col_hbm, w_hbm, x0_hbm, x1_hbm, x2_hbm, x3_hbm, z_hbm,
             o0, o1, o2, o3,
             rows_s, cols_s, w_s, xrows, accum,
             semg0, semg1, semsc0, semsc1, semst):
        cid = lax.axis_index("c")
        sid = lax.axis_index("s")
        semg = (semg0, semg1)
        semsc = (semsc0, semsc1)
        ebase = sid * EPT

        def stage_issue(mblk, pset):
            src_sl = pl.ds(ebase + mblk * SBLK, SBLK)
            dst_sl = pl.ds(pset * SBLK, SBLK)
            pltpu.async_copy(row_hbm.at[src_sl], rows_s.at[dst_sl], semst)
            pltpu.async_copy(col_hbm.at[src_sl], cols_s.at[dst_sl], semst)
            pltpu.async_copy(w_hbm.at[src_sl], w_s.at[dst_sl], semst)

        def stage_wait(pset):
            dst_sl = pl.ds(pset * SBLK, SBLK)
            pltpu.make_async_copy(row_hbm.at[pl.ds(0, SBLK)],
                                  rows_s.at[dst_sl], semst).wait()
            pltpu.make_async_copy(col_hbm.at[pl.ds(0, SBLK)],
                                  cols_s.at[dst_sl], semst).wait()
            pltpu.make_async_copy(w_hbm.at[pl.ds(0, SBLK)],
                                  w_s.at[dst_sl], semst).wait()

        def one_quarter(x_hbm, out_hbm):
            pltpu.sync_copy(z_hbm.at[pl.ds(sid * RPT, RPT)],
                            accum.at[pl.ds(sid * RPT, RPT)])
            stage_issue(0, 0)
            stage_wait(0)

            def issue_gather(jb, pset, b):
                idxc = cols_s.at[pl.ds(pset * SBLK + jb * C, C)]
                pltpu.async_copy(x_hbm.at[idxc], xrows.at[b], semg[b])

            def wait_gather(b):
                pltpu.make_async_copy(x_hbm.at[cols_s.at[pl.ds(0, C)]],
                                      xrows.at[b], semg[b]).wait()

            def wait_scatter(b):
                pltpu.make_async_copy(xrows.at[b],
                                      accum.at[rows_s.at[pl.ds(0, C)]],
                                      semsc[b]).wait()

            plsc.subcore_barrier()
            issue_gather(0, 0, 0)

            def outer(g, carry):
                for b in range(2):
                    j = 2 * g + b
                    m = j // BC
                    jb = j - m * BC
                    p = lax.rem(m, 2)

                    if b == 0:
                        @pl.when(j >= 1)
                        def _():
                            wait_scatter(1)
                    else:
                        wait_scatter(0)

                    @pl.when((jb == 0) & (m + 1 < NBLK))
                    def _():
                        stage_issue(m + 1, 1 - p)

                    @pl.when((jb == BC - 1) & (m + 1 < NBLK))
                    def _():
                        stage_wait(1 - p)

                    @pl.when(j + 1 < NCHQ)
                    def _():
                        j1 = j + 1
                        m1 = j1 // BC
                        jb1 = j1 - m1 * BC
                        p1 = lax.rem(m1, 2)
                        issue_gather(jb1, p1, 1 - b)

                    wait_gather(b)

                    for gg in range(C // L):
                        wg = w_s[pl.ds(p * SBLK + jb * C + gg * L, L)]
                        for e16 in range(L):
                            e = gg * L + e16
                            we = wg[e16]
                            for jj in range(NSLC):
                                xrows[b, e, pl.ds(jj * L, L)] = (
                                    we * xrows[b, e, pl.ds(jj * L, L)])
                    idxr = rows_s.at[pl.ds(p * SBLK + jb * C, C)]
                    pltpu.async_copy(xrows.at[b], accum.at[idxr],
                                     semsc[b], add=True)
                return carry

            lax.fori_loop(0, NCHQ // 2, outer, 0)
            wait_scatter((NCHQ - 1) % 2)
            plsc.subcore_barrier()
            pltpu.sync_copy(accum.at[pl.ds(sid * RPT, RPT)],
                            out_hbm.at[pl.ds(sid * RPT, RPT)])

        @pl.when(cid == 0)
        def _():
            one_quarter(x0_hbm, o0)
            one_quarter(x1_hbm, o1)

        @pl.when(cid == 1)
        def _():
            one_quarter(x2_hbm, o2)
            one_quarter(x3_hbm, o3)

    return pl.kernel(
        body,
        out_type=[jax.ShapeDtypeStruct((NP, DIM), jnp.float32)] * 4,
        mesh=mesh,
        compiler_params=pltpu.CompilerParams(needs_layout_passes=False),
        scratch_types=[
            pltpu.VMEM((2 * SBLK,), jnp.int32),
            pltpu.VMEM((2 * SBLK,), jnp.int32),
            pltpu.VMEM((2 * SBLK,), jnp.float32),
            pltpu.VMEM((2, C, DIM), jnp.float32),
            pltpu.VMEM_SHARED((NP, DIM), jnp.float32),
            pltpu.SemaphoreType.DMA,
            pltpu.SemaphoreType.DMA,
            pltpu.SemaphoreType.DMA,
            pltpu.SemaphoreType.DMA,
            pltpu.SemaphoreType.DMA,
        ],
    )(row, col, w, x0, x1, x2, x3, zeros)


def _finish_call(s, v, a0, a1, a2, a3, Wm, bm2):
    N, DIM = s.shape

    def body(s_ref, v_ref, a0_ref, a1_ref, a2_ref, a3_ref, wm_ref, bm_ref,
             so_ref, vo_ref):
        so_ref[...] = s_ref[...] + a0_ref[...]
        wm = wm_ref[...]
        for j, aj in enumerate((a1_ref, a2_ref, a3_ref)):
            vo_ref[:, j, :] = (
                v_ref[:, j, :]
                + jnp.dot(aj[...], wm, preferred_element_type=jnp.float32)
                + bm_ref[...])

    BLK = 1000
    grid = (N // BLK,)
    blk2d = pl.BlockSpec((BLK, DIM), lambda i: (i, 0))
    return pl.pallas_call(
        body,
        grid=grid,
        in_specs=[
            blk2d,
            pl.BlockSpec((BLK, 3, DIM), lambda i: (i, 0, 0)),
            blk2d, blk2d, blk2d, blk2d,
            pl.BlockSpec((DIM, DIM), lambda i: (0, 0)),
            pl.BlockSpec((1, DIM), lambda i: (0, 0)),
        ],
        out_specs=[
            blk2d,
            pl.BlockSpec((BLK, 3, DIM), lambda i: (i, 0, 0)),
        ],
        out_shape=[
            jax.ShapeDtypeStruct((N, DIM), jnp.float32),
            jax.ShapeDtypeStruct((N, 3, DIM), jnp.float32),
        ],
    )(s, v, a0, a1, a2, a3, Wm, bm2)


def kernel(s, v, edge_index, Wq, bq, Wkv, bkv, Wm, bm):
    N, DIM = s.shape
    row = edge_index[0]
    col = edge_index[1]
    q, k, x0, x1, x2, x3 = _proj_call(
        s, v, Wq, bq.reshape(1, -1), Wkv, bkv.reshape(1, -1))
    w = _edge_w_call(row, col, q, k)
    NP = ((N + 8 * NS - 1) // (8 * NS)) * (8 * NS)
    zeros = jnp.zeros((NP, DIM), jnp.float32)
    a0, a1, a2, a3 = _scatter_call(row, col, w, x0, x1, x2, x3, zeros)
    a0, a1, a2, a3 = (a[:N] for a in (a0, a1, a2, a3))
    return _finish_call(s, v, a0, a1, a2, a3, Wm, bm.reshape(1, -1))


# speedup vs baseline: 40.7439x; 1.4911x over previous
import jax
import jax.numpy as jnp
from jax import lax
from jax.experimental import pallas as pl
from jax.experimental.pallas import tpu as pltpu
from jax.experimental.pallas import tpu_sc as plsc

NC = 2
NS = 16
L = 16


def _proj_call(s, v, Wq, bq2, Wkv, bkv2):
    N, DIM = s.shape

    def body(s_ref, v_ref, wq_ref, bq_ref, wkv_ref, bkv_ref,
             q_ref, k_ref, x0_ref, x1_ref, x2_ref, x3_ref):
        sblk = s_ref[...]
        q_ref[...] = (
            jnp.dot(sblk, wq_ref[...], preferred_element_type=jnp.float32)
            + bq_ref[...])
        kv = (jnp.dot(sblk, wkv_ref[...], preferred_element_type=jnp.float32)
              + bkv_ref[...])
        k_ref[...] = kv[:, :DIM]
        x0_ref[...] = kv[:, DIM:2 * DIM]
        vv = kv[:, 2 * DIM:]
        x1_ref[...] = vv * v_ref[:, 0, :]
        x2_ref[...] = vv * v_ref[:, 1, :]
        x3_ref[...] = vv * v_ref[:, 2, :]

    BLK = 1000
    grid = (N // BLK,)
    out2d = pl.BlockSpec((BLK, DIM), lambda i: (i, 0))
    return pl.pallas_call(
        body,
        grid=grid,
        in_specs=[
            pl.BlockSpec((BLK, DIM), lambda i: (i, 0)),
            pl.BlockSpec((BLK, 3, DIM), lambda i: (i, 0, 0)),
            pl.BlockSpec((DIM, DIM), lambda i: (0, 0)),
            pl.BlockSpec((1, DIM), lambda i: (0, 0)),
            pl.BlockSpec((DIM, 3 * DIM), lambda i: (0, 0)),
            pl.BlockSpec((1, 3 * DIM), lambda i: (0, 0)),
        ],
        out_specs=[out2d] * 6,
        out_shape=[jax.ShapeDtypeStruct((N, DIM), jnp.float32)] * 6,
    )(s, v, Wq, bq2, Wkv, bkv2)


def _edge_w_call(row, col, q, k):
    (E,) = row.shape
    N, DIM = q.shape
    NW = NC * NS
    EPW = E // NW
    C = 80
    NCHUNK = EPW // C
    NSLC = DIM // L

    mesh = plsc.VectorSubcoreMesh(core_axis_name="c", subcore_axis_name="s",
                                  num_cores=NC, num_subcores=NS)

    def body(row_hbm, col_hbm, q_hbm, k_hbm, w_hbm,
             rows_v, cols_v, qrows, krows, wbuf,
             semg0, semg1, semg2, semw0, semw1, semw2):
        cid = lax.axis_index("c")
        sid = lax.axis_index("s")
        wid = sid * NC + cid
        base = wid * EPW
        lane = lax.iota(jnp.int32, L)
        semg = (semg0, semg1, semg2)
        semw = (semw0, semw1, semw2)

        pltpu.sync_copy(row_hbm.at[pl.ds(base, EPW)], rows_v)
        pltpu.sync_copy(col_hbm.at[pl.ds(base, EPW)], cols_v)

        def issue(j, b):
            idxr = rows_v.at[pl.ds(j * C, C)]
            idxc = cols_v.at[pl.ds(j * C, C)]
            pltpu.async_copy(q_hbm.at[idxr], qrows.at[b], semg[b])
            pltpu.async_copy(k_hbm.at[idxc], krows.at[b], semg[b])

        def wait_gather(b):
            dummy = rows_v.at[pl.ds(0, C)]
            pltpu.make_async_copy(q_hbm.at[dummy], qrows.at[b],
                                  semg[b]).wait()
            pltpu.make_async_copy(k_hbm.at[dummy], krows.at[b],
                                  semg[b]).wait()

        def wait_w(b):
            pltpu.make_async_copy(wbuf.at[b], w_hbm.at[pl.ds(base, C)],
                                  semw[b]).wait()

        def compute(b):
            def group(g, carry):
                wvec = jnp.zeros((L,), jnp.float32)
                for e16 in range(L):
                    e = g * L + e16
                    acc = qrows[b, e, pl.ds(0, L)] * krows[b, e, pl.ds(0, L)]
                    for jj in range(1, NSLC):
                        acc = acc + (qrows[b, e, pl.ds(jj * L, L)]
                                     * krows[b, e, pl.ds(jj * L, L)])
                    we = jnp.sum(acc, axis=0)
                    wvec = jnp.where(lane == e16, we, wvec)
                wbuf[b, pl.ds(g * L, L)] = wvec
                return carry

            lax.fori_loop(0, C // L, group, 0)

        issue(0, 0)
        issue(1, 1)

        def outer(g, carry):
            for b in range(3):
                j = 3 * g + b

                @pl.when(j < NCHUNK)
                def _():
                    @pl.when(j >= 3)
                    def _():
                        wait_w(b)

                    @pl.when(j + 2 < NCHUNK)
                    def _():
                        issue(j + 2, (b + 2) % 3)

                    wait_gather(b)
                    compute(b)
                    pltpu.async_copy(wbuf.at[b],
                                     w_hbm.at[pl.ds(base + j * C, C)],
                                     semw[b])
            return carry

        lax.fori_loop(0, (NCHUNK + 2) // 3, outer, 0)
        for b in range(3):
            wait_w(b)

    return pl.kernel(
        body,
        out_type=jax.ShapeDtypeStruct((E,), jnp.float32),
        mesh=mesh,
        compiler_params=pltpu.CompilerParams(needs_layout_passes=False),
        scratch_types=[
            pltpu.VMEM((EPW,), jnp.int32),
            pltpu.VMEM((EPW,), jnp.int32),
            pltpu.VMEM((3, C, DIM), jnp.float32),
            pltpu.VMEM((3, C, DIM), jnp.float32),
            pltpu.VMEM((3, C), jnp.float32),
        ] + [pltpu.SemaphoreType.DMA] * 6,
    )(row, col, q, k)


def _scatter_call(row, col, w, x0, x1, x2, x3, zeros):
    (E,) = col.shape
    N, DIM = x0.shape
    NP = zeros.shape[0]
    EPT = E // NS
    RPT = NP // NS
    C = 80
    NCHQ = EPT // C
    BC = 10
    SBLK = BC * C
    NBLK = NCHQ // BC
    NSLC = DIM // L

    mesh = plsc.VectorSubcoreMesh(core_axis_name="c", subcore_axis_name="s",
                                  num_cores=NC, num_subcores=NS)

    def body(row_hbm, col_hbm, w_hbm, x0_hbm, x1_hbm, x2_hbm, x3_hbm, z_hbm,
             o0, o1, o2, o3,
             rows_s, cols_s, w_s, xrows, accum,
             semg0, semg1, semg2, semg3, semsc0, semsc1, semsc2, semsc3,
             semst):
        cid = lax.axis_index("c")
        sid = lax.axis_index("s")
        semg = (semg0, semg1, semg2, semg3)
        semsc = (semsc0, semsc1, semsc2, semsc3)
        ebase = sid * EPT

        def stage_issue(mblk, pset):
            src_sl = pl.ds(ebase + mblk * SBLK, SBLK)
            dst_sl = pl.ds(pset * SBLK, SBLK)
            pltpu.async_copy(row_hbm.at[src_sl], rows_s.at[dst_sl], semst)
            pltpu.async_copy(col_hbm.at[src_sl], cols_s.at[dst_sl], semst)
            pltpu.async_copy(w_hbm.at[src_sl], w_s.at[dst_sl], semst)

        def stage_wait(pset):
            dst_sl = pl.ds(pset * SBLK, SBLK)
            pltpu.make_async_copy(row_hbm.at[pl.ds(0, SBLK)],
                                  rows_s.at[dst_sl], semst).wait()
            pltpu.make_async_copy(col_hbm.at[pl.ds(0, SBLK)],
                                  cols_s.at[dst_sl], semst).wait()
            pltpu.make_async_copy(w_hbm.at[pl.ds(0, SBLK)],
                                  w_s.at[dst_sl], semst).wait()

        def one_quarter(x_hbm, out_hbm):
            pltpu.sync_copy(z_hbm.at[pl.ds(sid * RPT, RPT)],
                            accum.at[pl.ds(sid * RPT, RPT)])
            stage_issue(0, 0)
            stage_wait(0)

            def issue_gather(jb, pset, b):
                idxc = cols_s.at[pl.ds(pset * SBLK + jb * C, C)]
                pltpu.async_copy(x_hbm.at[idxc], xrows.at[b], semg[b])

            def wait_gather(b):
                pltpu.make_async_copy(x_hbm.at[cols_s.at[pl.ds(0, C)]],
                                      xrows.at[b], semg[b]).wait()

            def wait_scatter(b):
                pltpu.make_async_copy(xrows.at[b],
                                      accum.at[rows_s.at[pl.ds(0, C)]],
                                      semsc[b]).wait()

            plsc.subcore_barrier()
            issue_gather(0, 0, 0)
            issue_gather(1, 0, 1)

            def outer(g, carry):
                for b in range(4):
                    j = 4 * g + b

                    @pl.when(j < NCHQ)
                    def _():
                        m = j // BC
                        jb = j - m * BC
                        p = lax.rem(m, 2)

                        @pl.when(j >= 2)
                        def _():
                            wait_scatter((b + 2) % 4)

                        @pl.when((jb == 2) & (m + 1 < NBLK))
                        def _():
                            stage_issue(m + 1, 1 - p)

                        @pl.when((jb == BC - 2) & (m + 1 < NBLK))
                        def _():
                            stage_wait(1 - p)

                        @pl.when(j + 2 < NCHQ)
                        def _():
                            j2 = j + 2
                            m2 = j2 // BC
                            jb2 = j2 - m2 * BC
                            p2 = lax.rem(m2, 2)
                            issue_gather(jb2, p2, (b + 2) % 4)

                        wait_gather(b)

                        wbase = p * SBLK + jb * C

                        def group(gg, carry):
                            wg = w_s[pl.ds(wbase + gg * L, L)]
                            for e16 in range(L):
                                we = wg[e16]
                                e = gg * L + e16
                                for jj in range(NSLC):
                                    xrows[b, e, pl.ds(jj * L, L)] = (
                                        we * xrows[b, e, pl.ds(jj * L, L)])
                            return carry

                        lax.fori_loop(0, C // L, group, 0)
                        idxr = rows_s.at[pl.ds(p * SBLK + jb * C, C)]
                        pltpu.async_copy(xrows.at[b], accum.at[idxr],
                                         semsc[b], add=True)
                return carry

            lax.fori_loop(0, (NCHQ + 3) // 4, outer, 0)
            wait_scatter((NCHQ - 2) % 4)
            wait_scatter((NCHQ - 1) % 4)
            plsc.subcore_barrier()
            pltpu.sync_copy(accum.at[pl.ds(sid * RPT, RPT)],
                            out_hbm.at[pl.ds(sid * RPT, RPT)])

        @pl.when(cid == 0)
        def _():
            one_quarter(x0_hbm, o0)
            one_quarter(x1_hbm, o1)

        @pl.when(cid == 1)
        def _():
            one_quarter(x2_hbm, o2)
            one_quarter(x3_hbm, o3)

    return pl.kernel(
        body,
        out_type=[jax.ShapeDtypeStruct((NP, DIM), jnp.float32)] * 4,
        mesh=mesh,
        compiler_params=pltpu.CompilerParams(needs_layout_passes=False),
        scratch_types=[
            pltpu.VMEM((2 * SBLK,), jnp.int32),
            pltpu.VMEM((2 * SBLK,), jnp.int32),
            pltpu.VMEM((2 * SBLK,), jnp.float32),
            pltpu.VMEM((4, C, DIM), jnp.float32),
            pltpu.VMEM_SHARED((NP, DIM), jnp.float32),
        ] + [pltpu.SemaphoreType.DMA] * 9,
    )(row, col, w, x0, x1, x2, x3, zeros)


def _finish_call(s, v, a0, a1, a2, a3, Wm, bm2):
    N, DIM = s.shape

    def body(s_ref, v_ref, a0_ref, a1_ref, a2_ref, a3_ref, wm_ref, bm_ref,
             so_ref, vo_ref):
        so_ref[...] = s_ref[...] + a0_ref[...]
        wm = wm_ref[...]
        for j, aj in enumerate((a1_ref, a2_ref, a3_ref)):
            vo_ref[:, j, :] = (
                v_ref[:, j, :]
                + jnp.dot(aj[...], wm, preferred_element_type=jnp.float32)
                + bm_ref[...])

    BLK = 1000
    grid = (N // BLK,)
    blk2d = pl.BlockSpec((BLK, DIM), lambda i: (i, 0))
    return pl.pallas_call(
        body,
        grid=grid,
        in_specs=[
            blk2d,
            pl.BlockSpec((BLK, 3, DIM), lambda i: (i, 0, 0)),
            blk2d, blk2d, blk2d, blk2d,
            pl.BlockSpec((DIM, DIM), lambda i: (0, 0)),
            pl.BlockSpec((1, DIM), lambda i: (0, 0)),
        ],
        out_specs=[
            blk2d,
            pl.BlockSpec((BLK, 3, DIM), lambda i: (i, 0, 0)),
        ],
        out_shape=[
            jax.ShapeDtypeStruct((N, DIM), jnp.float32),
            jax.ShapeDtypeStruct((N, 3, DIM), jnp.float32),
        ],
    )(s, v, a0, a1, a2, a3, Wm, bm2)


def kernel(s, v, edge_index, Wq, bq, Wkv, bkv, Wm, bm):
    N, DIM = s.shape
    row = edge_index[0]
    col = edge_index[1]
    q, k, x0, x1, x2, x3 = _proj_call(
        s, v, Wq, bq.reshape(1, -1), Wkv, bkv.reshape(1, -1))
    w = _edge_w_call(row, col, q, k)
    NP = ((N + 8 * NS - 1) // (8 * NS)) * (8 * NS)
    zeros = jnp.zeros((NP, DIM), jnp.float32)
    a0, a1, a2, a3 = _scatter_call(row, col, w, x0, x1, x2, x3, zeros)
    a0, a1, a2, a3 = (a[:N] for a in (a0, a1, a2, a3))
    return _finish_call(s, v, a0, a1, a2, a3, Wm, bm.reshape(1, -1))
